# baseline (speedup 1.0000x reference)
import jax
import jax.numpy as jnp
from jax import lax
from jax.experimental import pallas as pl
from jax.experimental.pallas import tpu as pltpu
from jax.experimental.pallas import tpu_sc as plsc

N_ROWS = 100_000
T_TRI = 1_000_000
L = 16
NC, NS = 2, 16
NW = NC * NS
CPT = 32_768
BLK = 2_048
GPB = BLK // L
NB = CPT // BLK
TAIL_TILE = 30
TAIL_START = 30 * CPT + 8 * BLK
TAIL_N = T_TRI - TAIL_START
HI_MASK = -65536


def _pack_table(embed):
    b16 = embed.astype(jnp.bfloat16)
    return jax.lax.bitcast_convert_type(
        jax.lax.bitcast_convert_type(b16, jnp.uint16), jnp.int32)


def _body(tab_hbm, trip_hbm, wt_hbm, ti_hbm, tj_hbm, tk_hbm,
          loss_hbm, viol_hbm,
          tab_v, ib0, ib1, jb0, jb1, kb0, kb1, wb0, wb1,
          tbi, tbj, tbk, st_l, st_v, sem0, sem1):
    c = lax.axis_index("c")
    s = lax.axis_index("s")
    w = s * NC + c
    base = w * CPT
    nb = jnp.clip((T_TRI - base) // BLK, 0, NB)

    def copies(b, ib, jb, kb, wb, sem):
        st = base + b * BLK
        return (
            pltpu.make_async_copy(
                trip_hbm.at[pl.ds(0, 1), pl.ds(st, BLK)], ib, sem),
            pltpu.make_async_copy(
                trip_hbm.at[pl.ds(1, 1), pl.ds(st, BLK)], jb, sem),
            pltpu.make_async_copy(
                trip_hbm.at[pl.ds(2, 1), pl.ds(st, BLK)], kb, sem),
            pltpu.make_async_copy(wt_hbm.at[pl.ds(st, BLK)], wb, sem),
        )

    def start_block(b, ib, jb, kb, wb, sem):
        for cp in copies(b, ib, jb, kb, wb, sem):
            cp.start()

    def wait_block(b, ib, jb, kb, wb, sem):
        for cp in copies(b, ib, jb, kb, wb, sem):
            cp.wait()

    @pl.when(nb > 0)
    def _prime():
        start_block(0, ib0, jb0, kb0, wb0, sem0)
        start_block(1, ib1, jb1, kb1, wb1, sem1)

    pltpu.sync_copy(tab_hbm, tab_v)

    def group(g, al_av, ib, jb, kb, wb):
        al, av = al_av
        ii = ib[0, pl.ds(g * L, L)]
        jj = jb[0, pl.ds(g * L, L)]
        kk = kb[0, pl.ds(g * L, L)]
        wi = plsc.load_gather(tab_v, [ii])
        wj = plsc.load_gather(tab_v, [jj])
        wk = plsc.load_gather(tab_v, [kk])
        xi = plsc.bitcast(wi << 16, jnp.float32)
        yi = plsc.bitcast(wi & HI_MASK, jnp.float32)
        xj = plsc.bitcast(wj << 16, jnp.float32)
        yj = plsc.bitcast(wj & HI_MASK, jnp.float32)
        xk = plsc.bitcast(wk << 16, jnp.float32)
        yk = plsc.bitcast(wk & HI_MASK, jnp.float32)
        dx1 = xi - xj
        dy1 = yi - yj
        dx2 = xi - xk
        dy2 = yi - yk
        dij = 1.0 + dx1 * dx1 + dy1 * dy1
        dik = 1.0 + dx2 * dx2 + dy2 * dy2
        ww = wb[pl.ds(g * L, L)]
        al = al + ww * dij / (dij + dik)
        av = av + jnp.where(dij > dik, 1.0, 0.0)
        return al, av

    def pair(i, al_av):
        b0 = 2 * i
        wait_block(b0, ib0, jb0, kb0, wb0, sem0)

        @pl.when(b0 + 2 < nb)
        def _():
            start_block(b0 + 2, ib0, jb0, kb0, wb0, sem0)

        al_av = lax.fori_loop(
            0, GPB, lambda g, cr: group(g, cr, ib0, jb0, kb0, wb0),
            al_av, unroll=16)
        wait_block(b0 + 1, ib1, jb1, kb1, wb1, sem1)

        @pl.when(b0 + 3 < nb)
        def _():
            start_block(b0 + 3, ib1, jb1, kb1, wb1, sem1)

        return lax.fori_loop(
            0, GPB, lambda g, cr: group(g, cr, ib1, jb1, kb1, wb1),
            al_av, unroll=16)

    zero = jnp.zeros((L,), jnp.float32)
    al, av = lax.fori_loop(0, nb // 2, pair, (zero, zero))
    st_l[...] = al
    st_v[...] = av

    @pl.when(w == TAIL_TILE)
    def _tail():
        pltpu.sync_copy(ti_hbm, tbi)
        pltpu.sync_copy(tj_hbm, tbj)
        pltpu.sync_copy(tk_hbm, tbk)
        pltpu.sync_copy(wt_hbm.at[pl.ds(TAIL_START, TAIL_N)],
                        wb0.at[pl.ds(0, TAIL_N)])
        tl, tv = lax.fori_loop(
            0, TAIL_N // L, lambda g, cr: group(g, cr, tbi, tbj, tbk, wb0),
            (st_l[...], st_v[...]))
        st_l[...] = tl
        st_v[...] = tv

    pltpu.sync_copy(st_l, loss_hbm.at[w])
    pltpu.sync_copy(st_v, viol_hbm.at[w])


def kernel(embed_init, triplets, weights):
    tab = _pack_table(embed_init)
    trip = triplets.astype(jnp.int32)
    trip_t = trip.T
    ti = trip[TAIL_START:, 0].reshape(1, TAIL_N)
    tj = trip[TAIL_START:, 1].reshape(1, TAIL_N)
    tk = trip[TAIL_START:, 2].reshape(1, TAIL_N)
    mesh = plsc.VectorSubcoreMesh(core_axis_name="c", subcore_axis_name="s",
                                  num_cores=NC, num_subcores=NS)
    blk_i32 = pltpu.VMEM((1, BLK), jnp.int32)
    blk_f32 = pltpu.VMEM((BLK,), jnp.float32)
    tail_i32 = pltpu.VMEM((1, TAIL_N), jnp.int32)
    fn = pl.kernel(
        _body,
        out_type=(jax.ShapeDtypeStruct((NW, L), jnp.float32),
                  jax.ShapeDtypeStruct((NW, L), jnp.float32)),
        mesh=mesh,
        compiler_params=pltpu.CompilerParams(needs_layout_passes=False),
        scratch_types=[
            pltpu.VMEM((N_ROWS,), jnp.int32),
            blk_i32, blk_i32, blk_i32, blk_i32, blk_i32, blk_i32,
            blk_f32, blk_f32,
            tail_i32, tail_i32, tail_i32,
            pltpu.VMEM((L,), jnp.float32),
            pltpu.VMEM((L,), jnp.float32),
            pltpu.SemaphoreType.DMA,
            pltpu.SemaphoreType.DMA,
        ],
    )
    loss_p, viol_p = fn(tab, trip_t, weights, ti, tj, tk)
    return jnp.sum(loss_p), jnp.sum(viol_p)


# speedup vs baseline: 1.4115x; 1.4115x over previous
import jax
import jax.numpy as jnp
from jax import lax
from jax.experimental import pallas as pl
from jax.experimental.pallas import tpu as pltpu
from jax.experimental.pallas import tpu_sc as plsc

N_ROWS = 100_000
T_TRI = 1_000_000
L = 16
NC, NS = 2, 16
NW = NC * NS
CPT = 32_768
BLK = 2_048
GPB = BLK // L
NB = CPT // BLK
TAIL_TILE = 30
TAIL_START = 30 * CPT + 8 * BLK
TAIL_N = T_TRI - TAIL_START
HI_MASK = -65536


def _pack_table(embed):
    b16 = embed.astype(jnp.bfloat16)
    return jax.lax.bitcast_convert_type(
        jax.lax.bitcast_convert_type(b16, jnp.uint16), jnp.int32)


def _body(tab_hbm, trip_hbm, wt_hbm, ti_hbm, tj_hbm, tk_hbm,
          loss_hbm, viol_hbm,
          tab_v, ib0, ib1, jb0, jb1, kb0, kb1, wb0, wb1,
          tbi, tbj, tbk, st_l, st_v, sem0, sem1):
    c = lax.axis_index("c")
    s = lax.axis_index("s")
    w = s * NC + c
    base = w * CPT
    nb = jnp.clip((T_TRI - base) // BLK, 0, NB)

    def copies(b, ib, jb, kb, wb, sem):
        st = base + b * BLK
        return (
            pltpu.make_async_copy(
                trip_hbm.at[pl.ds(0, 1), pl.ds(st, BLK)], ib, sem),
            pltpu.make_async_copy(
                trip_hbm.at[pl.ds(1, 1), pl.ds(st, BLK)], jb, sem),
            pltpu.make_async_copy(
                trip_hbm.at[pl.ds(2, 1), pl.ds(st, BLK)], kb, sem),
            pltpu.make_async_copy(wt_hbm.at[pl.ds(st, BLK)], wb, sem),
        )

    def start_block(b, ib, jb, kb, wb, sem):
        for cp in copies(b, ib, jb, kb, wb, sem):
            cp.start()

    def wait_block(b, ib, jb, kb, wb, sem):
        for cp in copies(b, ib, jb, kb, wb, sem):
            cp.wait()

    @pl.when(nb > 0)
    def _prime():
        start_block(0, ib0, jb0, kb0, wb0, sem0)
        start_block(1, ib1, jb1, kb1, wb1, sem1)

    pltpu.sync_copy(tab_hbm, tab_v)

    def group(g, al_av, ib, jb, kb, wb):
        al, av = al_av
        ii = ib[0, pl.ds(g * L, L)]
        jj = jb[0, pl.ds(g * L, L)]
        kk = kb[0, pl.ds(g * L, L)]
        wi = plsc.load_gather(tab_v, [ii])
        wj = plsc.load_gather(tab_v, [jj])
        wk = plsc.load_gather(tab_v, [kk])
        xi = plsc.bitcast(wi << 16, jnp.float32)
        yi = plsc.bitcast(wi & HI_MASK, jnp.float32)
        xj = plsc.bitcast(wj << 16, jnp.float32)
        yj = plsc.bitcast(wj & HI_MASK, jnp.float32)
        xk = plsc.bitcast(wk << 16, jnp.float32)
        yk = plsc.bitcast(wk & HI_MASK, jnp.float32)
        dx1 = xi - xj
        dy1 = yi - yj
        dx2 = xi - xk
        dy2 = yi - yk
        dij = 1.0 + dx1 * dx1 + dy1 * dy1
        dik = 1.0 + dx2 * dx2 + dy2 * dy2
        ww = wb[pl.ds(g * L, L)]
        al = al + ww * dij / (dij + dik)
        av = av + jnp.where(dij > dik, 1.0, 0.0)
        return al, av

    def pair(i, al_av):
        b0 = 2 * i
        wait_block(b0, ib0, jb0, kb0, wb0, sem0)

        @pl.when(b0 + 2 < nb)
        def _():
            start_block(b0 + 2, ib0, jb0, kb0, wb0, sem0)

        al_av = lax.fori_loop(
            0, GPB, lambda g, cr: group(g, cr, ib0, jb0, kb0, wb0),
            al_av, unroll=8)
        wait_block(b0 + 1, ib1, jb1, kb1, wb1, sem1)

        @pl.when(b0 + 3 < nb)
        def _():
            start_block(b0 + 3, ib1, jb1, kb1, wb1, sem1)

        return lax.fori_loop(
            0, GPB, lambda g, cr: group(g, cr, ib1, jb1, kb1, wb1),
            al_av, unroll=8)

    zero = jnp.zeros((L,), jnp.float32)
    al, av = lax.fori_loop(0, nb // 2, pair, (zero, zero))
    st_l[...] = al
    st_v[...] = av

    @pl.when(w == TAIL_TILE)
    def _tail():
        pltpu.sync_copy(ti_hbm, tbi)
        pltpu.sync_copy(tj_hbm, tbj)
        pltpu.sync_copy(tk_hbm, tbk)
        pltpu.sync_copy(wt_hbm.at[pl.ds(TAIL_START, TAIL_N)],
                        wb0.at[pl.ds(0, TAIL_N)])
        tl, tv = lax.fori_loop(
            0, TAIL_N // L, lambda g, cr: group(g, cr, tbi, tbj, tbk, wb0),
            (st_l[...], st_v[...]))
        st_l[...] = tl
        st_v[...] = tv

    pltpu.sync_copy(st_l, loss_hbm.at[w])
    pltpu.sync_copy(st_v, viol_hbm.at[w])


def kernel(embed_init, triplets, weights):
    tab = _pack_table(embed_init)
    trip = triplets.astype(jnp.int32)
    trip_t = trip.T
    ti = trip[TAIL_START:, 0].reshape(1, TAIL_N)
    tj = trip[TAIL_START:, 1].reshape(1, TAIL_N)
    tk = trip[TAIL_START:, 2].reshape(1, TAIL_N)
    mesh = plsc.VectorSubcoreMesh(core_axis_name="c", subcore_axis_name="s",
                                  num_cores=NC, num_subcores=NS)
    blk_i32 = pltpu.VMEM((1, BLK), jnp.int32)
    blk_f32 = pltpu.VMEM((BLK,), jnp.float32)
    tail_i32 = pltpu.VMEM((1, TAIL_N), jnp.int32)
    fn = pl.kernel(
        _body,
        out_type=(jax.ShapeDtypeStruct((NW, L), jnp.float32),
                  jax.ShapeDtypeStruct((NW, L), jnp.float32)),
        mesh=mesh,
        compiler_params=pltpu.CompilerParams(needs_layout_passes=False),
        scratch_types=[
            pltpu.VMEM((N_ROWS,), jnp.int32),
            blk_i32, blk_i32, blk_i32, blk_i32, blk_i32, blk_i32,
            blk_f32, blk_f32,
            tail_i32, tail_i32, tail_i32,
            pltpu.VMEM((L,), jnp.float32),
            pltpu.VMEM((L,), jnp.float32),
            pltpu.SemaphoreType.DMA,
            pltpu.SemaphoreType.DMA,
        ],
    )
    loss_p, viol_p = fn(tab, trip_t, weights, ti, tj, tk)
    return jnp.sum(loss_p), jnp.sum(viol_p)


# speedup vs baseline: 1.4149x; 1.0024x over previous
import jax
import jax.numpy as jnp
from jax import lax
from jax.experimental import pallas as pl
from jax.experimental.pallas import tpu as pltpu
from jax.experimental.pallas import tpu_sc as plsc

N_ROWS = 100_000
T_TRI = 1_000_000
L = 16
NC, NS = 2, 16
NW = NC * NS
CPT = 32_768
BLK = 2_048
GPB = BLK // L
NB = CPT // BLK
TAIL_TILE = 30
TAIL_START = 30 * CPT + 8 * BLK
TAIL_N = T_TRI - TAIL_START
HI_MASK = -65536


def _pack_table(embed):
    b16 = embed.astype(jnp.bfloat16)
    return jax.lax.bitcast_convert_type(
        jax.lax.bitcast_convert_type(b16, jnp.uint16), jnp.int32)


def _body(tab_hbm, trip_hbm, wt_hbm, ti_hbm, tj_hbm, tk_hbm,
          loss_hbm, viol_hbm,
          tab_v, ib0, ib1, jb0, jb1, kb0, kb1, wb0, wb1,
          tbi, tbj, tbk, st_l, st_v, sem0, sem1):
    c = lax.axis_index("c")
    s = lax.axis_index("s")
    w = s * NC + c
    base = w * CPT
    nb = jnp.clip((T_TRI - base) // BLK, 0, NB)

    def copies(b, ib, jb, kb, wb, sem):
        st = base + b * BLK
        return (
            pltpu.make_async_copy(
                trip_hbm.at[pl.ds(0, 1), pl.ds(st, BLK)], ib, sem),
            pltpu.make_async_copy(
                trip_hbm.at[pl.ds(1, 1), pl.ds(st, BLK)], jb, sem),
            pltpu.make_async_copy(
                trip_hbm.at[pl.ds(2, 1), pl.ds(st, BLK)], kb, sem),
            pltpu.make_async_copy(wt_hbm.at[pl.ds(st, BLK)], wb, sem),
        )

    def start_block(b, ib, jb, kb, wb, sem):
        for cp in copies(b, ib, jb, kb, wb, sem):
            cp.start()

    def wait_block(b, ib, jb, kb, wb, sem):
        for cp in copies(b, ib, jb, kb, wb, sem):
            cp.wait()

    @pl.when(nb > 0)
    def _prime():
        start_block(0, ib0, jb0, kb0, wb0, sem0)
        start_block(1, ib1, jb1, kb1, wb1, sem1)

    pltpu.sync_copy(tab_hbm, tab_v)

    def group(g, al_av, ib, jb, kb, wb):
        al, av = al_av
        ii = ib[0, pl.ds(g * L, L)]
        jj = jb[0, pl.ds(g * L, L)]
        kk = kb[0, pl.ds(g * L, L)]
        wi = plsc.load_gather(tab_v, [ii])
        wj = plsc.load_gather(tab_v, [jj])
        wk = plsc.load_gather(tab_v, [kk])
        xi = plsc.bitcast(wi << 16, jnp.float32)
        yi = plsc.bitcast(wi & HI_MASK, jnp.float32)
        xj = plsc.bitcast(wj << 16, jnp.float32)
        yj = plsc.bitcast(wj & HI_MASK, jnp.float32)
        xk = plsc.bitcast(wk << 16, jnp.float32)
        yk = plsc.bitcast(wk & HI_MASK, jnp.float32)
        dx1 = xi - xj
        dy1 = yi - yj
        dx2 = xi - xk
        dy2 = yi - yk
        dij = 1.0 + dx1 * dx1 + dy1 * dy1
        dik = 1.0 + dx2 * dx2 + dy2 * dy2
        ww = wb[pl.ds(g * L, L)]
        al = al + ww * dij / (dij + dik)
        av = av + jnp.where(dij > dik, 1.0, 0.0)
        return al, av

    def pair(i, al_av):
        b0 = 2 * i
        wait_block(b0, ib0, jb0, kb0, wb0, sem0)

        @pl.when(b0 + 2 < nb)
        def _():
            start_block(b0 + 2, ib0, jb0, kb0, wb0, sem0)

        al_av = lax.fori_loop(
            0, GPB, lambda g, cr: group(g, cr, ib0, jb0, kb0, wb0),
            al_av, unroll=4)
        wait_block(b0 + 1, ib1, jb1, kb1, wb1, sem1)

        @pl.when(b0 + 3 < nb)
        def _():
            start_block(b0 + 3, ib1, jb1, kb1, wb1, sem1)

        return lax.fori_loop(
            0, GPB, lambda g, cr: group(g, cr, ib1, jb1, kb1, wb1),
            al_av, unroll=4)

    zero = jnp.zeros((L,), jnp.float32)
    al, av = lax.fori_loop(0, nb // 2, pair, (zero, zero))
    st_l[...] = al
    st_v[...] = av

    @pl.when(w == TAIL_TILE)
    def _tail():
        pltpu.sync_copy(ti_hbm, tbi)
        pltpu.sync_copy(tj_hbm, tbj)
        pltpu.sync_copy(tk_hbm, tbk)
        pltpu.sync_copy(wt_hbm.at[pl.ds(TAIL_START, TAIL_N)],
                        wb0.at[pl.ds(0, TAIL_N)])
        tl, tv = lax.fori_loop(
            0, TAIL_N // L, lambda g, cr: group(g, cr, tbi, tbj, tbk, wb0),
            (st_l[...], st_v[...]))
        st_l[...] = tl
        st_v[...] = tv

    pltpu.sync_copy(st_l, loss_hbm.at[w])
    pltpu.sync_copy(st_v, viol_hbm.at[w])


def kernel(embed_init, triplets, weights):
    tab = _pack_table(embed_init)
    trip = triplets.astype(jnp.int32)
    trip_t = trip.T
    ti = trip[TAIL_START:, 0].reshape(1, TAIL_N)
    tj = trip[TAIL_START:, 1].reshape(1, TAIL_N)
    tk = trip[TAIL_START:, 2].reshape(1, TAIL_N)
    mesh = plsc.VectorSubcoreMesh(core_axis_name="c", subcore_axis_name="s",
                                  num_cores=NC, num_subcores=NS)
    blk_i32 = pltpu.VMEM((1, BLK), jnp.int32)
    blk_f32 = pltpu.VMEM((BLK,), jnp.float32)
    tail_i32 = pltpu.VMEM((1, TAIL_N), jnp.int32)
    fn = pl.kernel(
        _body,
        out_type=(jax.ShapeDtypeStruct((NW, L), jnp.float32),
                  jax.ShapeDtypeStruct((NW, L), jnp.float32)),
        mesh=mesh,
        compiler_params=pltpu.CompilerParams(needs_layout_passes=False),
        scratch_types=[
            pltpu.VMEM((N_ROWS,), jnp.int32),
            blk_i32, blk_i32, blk_i32, blk_i32, blk_i32, blk_i32,
            blk_f32, blk_f32,
            tail_i32, tail_i32, tail_i32,
            pltpu.VMEM((L,), jnp.float32),
            pltpu.VMEM((L,), jnp.float32),
            pltpu.SemaphoreType.DMA,
            pltpu.SemaphoreType.DMA,
        ],
    )
    loss_p, viol_p = fn(tab, trip_t, weights, ti, tj, tk)
    return jnp.sum(loss_p), jnp.sum(viol_p)
